# Initial kernel scaffold; baseline (speedup 1.0000x reference)
import jax
import jax.numpy as jnp
from jax.experimental import pallas as pl


def kernel(fs0, fs1, fs2, fs3, proposals):
    raise NotImplementedError("write your pallas kernel here")


# speedup vs baseline: 27.1894x; 27.1894x over previous
import functools

import jax
import jax.numpy as jnp
import numpy as np
from jax import lax
from jax.experimental import pallas as pl
from jax.experimental.pallas import tpu as pltpu
from jax.experimental.pallas import tpu_sc as plsc

_CS = 7
_NPTS = _CS * _CS
_NCOL = 4 * _NPTS
_NCOLP = 224
_NW = 32
_G0, _G1 = 112, 96
_NROWS = _G0 + _G1

_col = np.arange(_NCOLP)
_p = _col // 4
_c = _col % 4
_TY = ((_p // _CS + 0.5) / _CS).astype(np.float32)[None, :]
_TX = ((_p % _CS + 0.5) / _CS).astype(np.float32)[None, :]
_CY = (_c >> 1).astype(np.int32)[None, :]
_CX = (_c & 1).astype(np.int32)[None, :]

_LVL_INV = (0.25, 0.125, 0.0625, 0.03125)
_LVL_W = (128, 64, 32, 16)
_LVL_OFF = (0, 16384, 20480, 21504)


def _phase1_body(b_ref, tx_ref, ty_ref, cx_ref, cy_ref, idx_ref, w_ref):
    b = b_ref[:]
    x0 = b[:, 0:1]
    y0 = b[:, 1:2]
    x1 = b[:, 2:3]
    y1 = b[:, 3:4]
    size = jnp.sqrt((x1 - x0) * (y1 - y0))
    lvl = ((size > 12.0).astype(jnp.int32)
           + (size > 24.0).astype(jnp.int32)
           + (size > 48.0).astype(jnp.int32))

    def sel(vals, dtype):
        r = jnp.full(lvl.shape, vals[3], dtype)
        for l in (2, 1, 0):
            r = jnp.where(lvl == l, jnp.asarray(vals[l], dtype), r)
        return r

    inv = sel(_LVL_INV, jnp.float32)
    wl = sel(_LVL_W, jnp.int32)
    off = sel(_LVL_OFF, jnp.int32)

    txc = tx_ref[:]
    tyc = ty_ref[:]
    cxi = cx_ref[:]
    cyi = cy_ref[:]
    cxf = cxi.astype(jnp.float32)
    cyf = cyi.astype(jnp.float32)

    bx0 = x0 * inv
    bx1 = x1 * inv
    by0 = y0 * inv
    by1 = y1 * inv
    xs = bx0 + (bx1 - bx0) * txc
    ys = by0 + (by1 - by0) * tyc
    xf = jnp.floor(xs)
    yf = jnp.floor(ys)
    fx = xs - xf
    fy = ys - yf
    xi = jnp.clip(xf.astype(jnp.int32) + cxi, 0, wl - 1)
    yi = jnp.clip(yf.astype(jnp.int32) + cyi, 0, wl - 1)
    wx = (1.0 - fx) + cxf * (2.0 * fx - 1.0)
    wy = (1.0 - fy) + cyf * (2.0 * fy - 1.0)
    idx_ref[:] = off + yi * wl + xi
    w_ref[:] = wy * wx


def _phase1(boxes):
    npad = boxes.shape[0]
    return pl.pallas_call(
        _phase1_body,
        out_shape=[
            jax.ShapeDtypeStruct((npad, _NCOLP), jnp.int32),
            jax.ShapeDtypeStruct((npad, _NCOLP), jnp.float32),
        ],
    )(boxes, jnp.asarray(_TX), jnp.asarray(_TY),
      jnp.asarray(_CX), jnp.asarray(_CY))


def _make_crop_sc(n, c):
    q, r = divmod(n, _NW)
    nmax = q + 1 if r else q
    npairs = (nmax + 1) // 2
    cvregs = c // 16
    mesh = plsc.VectorSubcoreMesh(core_axis_name="c", subcore_axis_name="s")

    @functools.partial(
        pl.kernel,
        mesh=mesh,
        out_type=jax.ShapeDtypeStruct((n, c, _NPTS), jnp.float32),
        compiler_params=pltpu.CompilerParams(
            use_tc_tiling_on_sc=False, needs_layout_passes=False),
        scratch_types=[
            pltpu.VMEM((nmax, _NCOLP), jnp.int32),
            pltpu.VMEM((nmax, _NCOLP), jnp.float32),
            pltpu.VMEM((_NROWS, c), jnp.float32),
            pltpu.VMEM((_NROWS, c), jnp.float32),
            pltpu.VMEM((c, _NPTS), jnp.float32),
            pltpu.VMEM((c, _NPTS), jnp.float32),
            pltpu.SemaphoreType.DMA,
            pltpu.SemaphoreType.DMA,
            pltpu.SemaphoreType.DMA,
            pltpu.SemaphoreType.DMA,
        ],
    )
    def crop_sc(table_hbm, idx_hbm, w_hbm, out_hbm,
                idxv, wv, rows0, rows1, ob0, ob1, sg0, sg1, ss0, ss1):
        wid = lax.axis_index("s") * 2 + lax.axis_index("c")
        nloc = jnp.where(wid < r, q + 1, q) if r else q
        base = (jnp.where(wid < r, (q + 1) * wid, r * (q + 1) + q * (wid - r))
                if r else q * wid)

        pltpu.sync_copy(idx_hbm.at[pl.ds(base, nmax)], idxv)
        pltpu.sync_copy(w_hbm.at[pl.ds(base, nmax)], wv)

        def gcopies(p, rb, sem):
            return (
                pltpu.make_async_copy(
                    table_hbm.at[idxv.at[p, pl.ds(0, _G0)]],
                    rb.at[pl.ds(0, _G0)], sem),
                pltpu.make_async_copy(
                    table_hbm.at[idxv.at[p, pl.ds(_G0, _G1)]],
                    rb.at[pl.ds(_G0, _G1)], sem),
            )

        def gstart(p, rb, sem):
            for cp in gcopies(p, rb, sem):
                cp.start()

        def gwait(p, rb, sem):
            for cp in gcopies(p, rb, sem):
                cp.wait()

        def scopy(p, ob, sem):
            return pltpu.make_async_copy(ob, out_hbm.at[base + p], sem)

        iota = lax.iota(jnp.int32, 16)
        d0s = [iota + cv * 16 for cv in range(cvregs)]

        def compute(p, rb, ob):
            def one_point(j, wvec, wbase):
                cb = 4 * j
                w0 = wvec[wbase]
                w1 = wvec[wbase + 1]
                w2 = wvec[wbase + 2]
                w3 = wvec[wbase + 3]
                jv = jnp.full((16,), 0, jnp.int32) + j
                for cv in range(cvregs):
                    s = pl.ds(cv * 16, 16)
                    acc = ((w0 * rb[cb, s] + w1 * rb[cb + 1, s])
                           + (w2 * rb[cb + 2, s] + w3 * rb[cb + 3, s]))
                    plsc.store_scatter(ob, [d0s[cv], jv], acc)

            def pt_pair(jj, carry):
                wvec = wv[p, pl.ds(8 * jj, 16)]
                one_point(2 * jj, wvec, 0)
                one_point(2 * jj + 1, wvec, 4)
                return carry

            lax.fori_loop(0, (_NPTS - 1) // 2, pt_pair, 0)
            one_point(_NPTS - 1, wv[p, pl.ds(4 * (_NPTS - 1), 16)], 0)

        gstart(0, rows0, sg0)

        @pl.when(nloc > 1)
        def _():
            gstart(1, rows1, sg1)

        def pair(iq, carry):
            for b, rb, ob, sg, ss in ((0, rows0, ob0, sg0, ss0),
                                      (1, rows1, ob1, sg1, ss1)):
                p = 2 * iq + b

                @pl.when(p < nloc)
                def _():
                    gwait(p, rb, sg)

                    @pl.when(p >= 2)
                    def _():
                        scopy(p - 2, ob, ss).wait()

                    compute(p, rb, ob)
                    scopy(p, ob, ss).start()

                    @pl.when(p + 2 < nloc)
                    def _():
                        gstart(p + 2, rb, sg)

            return carry

        lax.fori_loop(0, npairs, pair, 0)

        pe = ((nloc - 1) // 2) * 2
        po = ((nloc - 2) // 2) * 2 + 1
        scopy(pe, ob0, ss0).wait()

        @pl.when(nloc > 1)
        def _():
            scopy(po, ob1, ss1).wait()

    return crop_sc


def kernel(fs0, fs1, fs2, fs3, proposals):
    n = proposals.shape[0]
    c = fs0.shape[1]
    parts = []
    for f in (fs0, fs1, fs2, fs3):
        h, w = f.shape[2], f.shape[3]
        parts.append(jnp.transpose(f[0], (1, 2, 0)).reshape(h * w, c))
    table = jnp.concatenate(parts, axis=0)

    q, r = divmod(n, _NW)
    npad = _NW * (q + 1 if r else q)
    boxes = proposals[:, 1:5]
    if npad > n:
        boxes = jnp.concatenate(
            [boxes, jnp.zeros((npad - n, 4), jnp.float32)], axis=0)
    idx, wgt = _phase1(boxes)
    out = _make_crop_sc(n, c)(table, idx, wgt)
    return out.reshape(n, c, _CS, _CS)
